# baseline (speedup 1.0000x reference)
import jax
import jax.numpy as jnp
from jax.experimental import pallas as pl

M = 10000


def _bn(x, g, b, eps):
    m = jnp.mean(x, axis=0)
    v = jnp.var(x, axis=0)
    return (x - m) / jnp.sqrt(v + eps) * g + b


def _mm_body(x_ref, w_ref, b_ref, o_ref):
    o_ref[...] = jnp.dot(x_ref[...], w_ref[...],
                         preferred_element_type=jnp.float32) + b_ref[...]


def _matmul(x, w, b):
    n, k = x.shape
    ko, d = w.shape
    blk = 1000
    return pl.pallas_call(
        _mm_body,
        grid=(n // blk,),
        in_specs=[
            pl.BlockSpec((blk, k), lambda i: (i, 0)),
            pl.BlockSpec((k, d), lambda i: (0, 0)),
            pl.BlockSpec((d,), lambda i: (0,)),
        ],
        out_specs=pl.BlockSpec((blk, d), lambda i: (i, 0)),
        out_shape=jax.ShapeDtypeStruct((n, d), jnp.float32),
    )(x, w, b)


def kernel(inputs, unq_inv, W_lin, b_lin, g_norm, be_norm, W_w1, b_w1, g_w, be_w, W_w2, b_w2, W_pm, b_pm, g_pm, be_pm, W_pw, b_pw, g_lk, be_lk):
    counts = jax.ops.segment_sum(jnp.ones(inputs.shape[0], jnp.float32), unq_inv, num_segments=M)
    x = jax.nn.relu(_bn(_matmul(inputs, W_lin, b_lin), g_norm, be_norm, 1e-3))
    h = jax.nn.relu(_bn(_matmul(x, W_w1, b_w1), g_w, be_w, 1e-5))
    weight = h @ W_w2 + b_w2
    wmax = jax.ops.segment_max(weight, unq_inv, num_segments=M)
    wmax = jnp.where(counts[:, None] > 0, wmax, 0.0)
    e = jnp.exp(weight - wmax[unq_inv])
    denom = jax.ops.segment_sum(e, unq_inv, num_segments=M)[unq_inv]
    soft_weight = e / denom
    weight_x = soft_weight * x
    seg_sum = jax.ops.segment_sum(weight_x, unq_inv, num_segments=M)
    mean_feat = seg_sum / jnp.clip(counts, 1.0, None)[:, None]
    pxyz = _bn(_matmul(x, W_pm, b_pm), g_pm, be_pm, 1e-3)
    pw = _matmul(pxyz, W_pw, b_pw)
    fs = pxyz * jnp.sin(pw)
    fc = pxyz * jnp.cos(pw)
    add_s = jax.ops.segment_sum(fs, unq_inv, num_segments=M)
    add_c = jax.ops.segment_sum(fc, unq_inv, num_segments=M)
    feat = _bn(mean_feat @ W_pm + b_pm, g_pm, be_pm, 1e-3)
    mw = feat @ W_pw + b_pw
    cs = feat * jnp.sin(mw)
    cc = feat * jnp.cos(mw)
    add_s = add_s + cs
    add_c = add_c + cc
    final = add_s * cs + add_c * cc
    link_feat = jax.nn.relu(_bn(final, g_lk, be_lk, 1e-3))
    mean_feat = (mean_feat + link_feat) / 2.0
    xmax = jax.ops.segment_max(x, unq_inv, num_segments=M)
    xmax = jnp.where(counts[:, None] > 0, xmax, 0.0)
    return (mean_feat + xmax) / 2.0


# speedup vs baseline: 1.1372x; 1.1372x over previous
import functools

import jax
import jax.numpy as jnp
from jax import lax
from jax.experimental import pallas as pl
from jax.experimental.pallas import tpu as pltpu

M = 10000
N = 160000
D = 256
BLK = 1000
MBLK = 1000


def _stats(sums_ref, n, eps, g, b):
    mean = sums_ref[0:1, :] / n
    var = sums_ref[1:2, :] / n - mean * mean
    scale = g * lax.rsqrt(var + eps)
    shift = b - mean * scale
    return scale, shift


def _acc_stats(ref, val):
    i = pl.program_id(0)

    @pl.when(i == 0)
    def _():
        ref[...] = jnp.zeros_like(ref)

    s = jnp.sum(val, axis=0, keepdims=True)
    sq = jnp.sum(val * val, axis=0, keepdims=True)
    ref[0:2, :] += jnp.concatenate([s, sq], axis=0)


def _k1_body(inp_ref, w_ref, b_ref, a_ref, s1_ref):
    a = jnp.dot(inp_ref[...], w_ref[...], preferred_element_type=jnp.float32) + b_ref[...]
    a_ref[...] = a
    _acc_stats(s1_ref, a)


def _k1(inputs, W_lin, b_lin):
    return pl.pallas_call(
        _k1_body,
        grid=(N // BLK,),
        in_specs=[
            pl.BlockSpec((BLK, 128), lambda i: (i, 0)),
            pl.BlockSpec((128, D), lambda i: (0, 0)),
            pl.BlockSpec((1, D), lambda i: (0, 0)),
        ],
        out_specs=[
            pl.BlockSpec((BLK, D), lambda i: (i, 0)),
            pl.BlockSpec((8, D), lambda i: (0, 0)),
        ],
        out_shape=[
            jax.ShapeDtypeStruct((N, D), jnp.float32),
            jax.ShapeDtypeStruct((8, D), jnp.float32),
        ],
    )(inputs, W_lin, b_lin)


def _k2_body(a_ref, s1_ref, g1_ref, b1_ref, ww1_ref, bw1_ref, wpm_ref, bpm_ref,
             x_ref, bb_ref, cc_ref, s2_ref, s3_ref):
    scale, shift = _stats(s1_ref, N, 1e-3, g1_ref[...], b1_ref[...])
    x = jax.nn.relu(a_ref[...] * scale + shift)
    x_ref[...] = x
    bmat = jnp.dot(x, ww1_ref[...], preferred_element_type=jnp.float32) + bw1_ref[...]
    cmat = jnp.dot(x, wpm_ref[...], preferred_element_type=jnp.float32) + bpm_ref[...]
    bb_ref[...] = bmat
    cc_ref[...] = cmat
    _acc_stats(s2_ref, bmat)
    _acc_stats(s3_ref, cmat)


def _k2(A, s1, g1, b1, Ww1, bw1, Wpm, bpm):
    full = pl.BlockSpec((D, D), lambda i: (0, 0))
    vec = pl.BlockSpec((1, D), lambda i: (0, 0))
    st = pl.BlockSpec((8, D), lambda i: (0, 0))
    blk = pl.BlockSpec((BLK, D), lambda i: (i, 0))
    return pl.pallas_call(
        _k2_body,
        grid=(N // BLK,),
        in_specs=[blk, st, vec, vec, full, vec, full, vec],
        out_specs=[blk, blk, blk, st, st],
        out_shape=[
            jax.ShapeDtypeStruct((N, D), jnp.float32),
            jax.ShapeDtypeStruct((N, D), jnp.float32),
            jax.ShapeDtypeStruct((N, D), jnp.float32),
            jax.ShapeDtypeStruct((8, D), jnp.float32),
            jax.ShapeDtypeStruct((8, D), jnp.float32),
        ],
    )(A, s1, g1, b1, Ww1, bw1, Wpm, bpm)


def _k3a_body(bb_ref, s2_ref, g2_ref, b2_ref, ww2_ref, bw2_ref, w_ref):
    scale, shift = _stats(s2_ref, N, 1e-5, g2_ref[...], b2_ref[...])
    h = jax.nn.relu(bb_ref[...] * scale + shift)
    w_ref[...] = jnp.dot(h, ww2_ref[...], preferred_element_type=jnp.float32) + bw2_ref[...]


def _k3a(B, s2, g2, b2, Ww2, bw2):
    vec = pl.BlockSpec((1, D), lambda i: (0, 0))
    st = pl.BlockSpec((8, D), lambda i: (0, 0))
    blk = pl.BlockSpec((BLK, D), lambda i: (i, 0))
    return pl.pallas_call(
        _k3a_body,
        grid=(N // BLK,),
        in_specs=[blk, st, vec, vec,
                  pl.BlockSpec((D, 1), lambda i: (0, 0)),
                  pl.BlockSpec((1, 1), lambda i: (0, 0))],
        out_specs=pl.BlockSpec((BLK, 1), lambda i: (i, 0)),
        out_shape=jax.ShapeDtypeStruct((N, 1), jnp.float32),
    )(B, s2, g2, b2, Ww2, bw2)


def _k3b_body(c_ref, x_ref, e_ref, s3_ref, g3_ref, b3_ref, wpw_ref, bpw_ref,
              fs_ref, fc_ref, ex_ref):
    scale, shift = _stats(s3_ref, N, 1e-3, g3_ref[...], b3_ref[...])
    pxyz = c_ref[...] * scale + shift
    pw = jnp.dot(pxyz, wpw_ref[...], preferred_element_type=jnp.float32) + bpw_ref[...]
    fs_ref[...] = pxyz * jnp.sin(pw)
    fc_ref[...] = pxyz * jnp.cos(pw)
    ex_ref[...] = x_ref[...] * e_ref[...]


def _k3b(C, x, e, s3, g3, b3, Wpw, bpw):
    vec = pl.BlockSpec((1, D), lambda i: (0, 0))
    st = pl.BlockSpec((8, D), lambda i: (0, 0))
    blk = pl.BlockSpec((BLK, D), lambda i: (i, 0))
    return pl.pallas_call(
        _k3b_body,
        grid=(N // BLK,),
        in_specs=[blk, blk, pl.BlockSpec((BLK, 1), lambda i: (i, 0)), st, vec, vec,
                  pl.BlockSpec((D, D), lambda i: (0, 0)), vec],
        out_specs=[blk, blk, blk],
        out_shape=[jax.ShapeDtypeStruct((N, D), jnp.float32)] * 3,
    )(C, x, e, s3, g3, b3, Wpw, bpw)


def _k4_body(se_ref, inv_ref, wpm_ref, bpm_ref, mf_ref, d_ref, s4_ref):
    mf = se_ref[...] * inv_ref[...]
    mf_ref[...] = mf
    d = jnp.dot(mf, wpm_ref[...], preferred_element_type=jnp.float32) + bpm_ref[...]
    d_ref[...] = d
    _acc_stats(s4_ref, d)


def _k4(sum_ex, inv_scale, Wpm, bpm):
    vec = pl.BlockSpec((1, D), lambda i: (0, 0))
    st = pl.BlockSpec((8, D), lambda i: (0, 0))
    blk = pl.BlockSpec((MBLK, D), lambda i: (i, 0))
    return pl.pallas_call(
        _k4_body,
        grid=(M // MBLK,),
        in_specs=[blk, pl.BlockSpec((MBLK, 1), lambda i: (i, 0)),
                  pl.BlockSpec((D, D), lambda i: (0, 0)), vec],
        out_specs=[blk, blk, st],
        out_shape=[
            jax.ShapeDtypeStruct((M, D), jnp.float32),
            jax.ShapeDtypeStruct((M, D), jnp.float32),
            jax.ShapeDtypeStruct((8, D), jnp.float32),
        ],
    )(sum_ex, inv_scale, Wpm, bpm)


def _k5_body(d_ref, s4_ref, g3_ref, b3_ref, wpw_ref, bpw_ref, as_ref, ac_ref,
             fin_ref, s5_ref):
    scale, shift = _stats(s4_ref, M, 1e-3, g3_ref[...], b3_ref[...])
    feat = d_ref[...] * scale + shift
    mw = jnp.dot(feat, wpw_ref[...], preferred_element_type=jnp.float32) + bpw_ref[...]
    cs = feat * jnp.sin(mw)
    cc = feat * jnp.cos(mw)
    final = (as_ref[...] + cs) * cs + (ac_ref[...] + cc) * cc
    fin_ref[...] = final
    _acc_stats(s5_ref, final)


def _k5(Dm, s4, g3, b3, Wpw, bpw, add_s, add_c):
    vec = pl.BlockSpec((1, D), lambda i: (0, 0))
    st = pl.BlockSpec((8, D), lambda i: (0, 0))
    blk = pl.BlockSpec((MBLK, D), lambda i: (i, 0))
    return pl.pallas_call(
        _k5_body,
        grid=(M // MBLK,),
        in_specs=[blk, st, vec, vec, pl.BlockSpec((D, D), lambda i: (0, 0)), vec,
                  blk, blk],
        out_specs=[blk, st],
        out_shape=[
            jax.ShapeDtypeStruct((M, D), jnp.float32),
            jax.ShapeDtypeStruct((8, D), jnp.float32),
        ],
    )(Dm, s4, g3, b3, Wpw, bpw, add_s, add_c)


def _k6_body(fin_ref, s5_ref, g5_ref, b5_ref, mf_ref, xm_ref, cp_ref, out_ref):
    scale, shift = _stats(s5_ref, M, 1e-3, g5_ref[...], b5_ref[...])
    link = jax.nn.relu(fin_ref[...] * scale + shift)
    xmax = jnp.where(cp_ref[...] > 0, xm_ref[...], 0.0)
    out_ref[...] = ((mf_ref[...] + link) * 0.5 + xmax) * 0.5


def _k6(final, s5, g5, b5, mean_feat, xmax, cntpos):
    vec = pl.BlockSpec((1, D), lambda i: (0, 0))
    st = pl.BlockSpec((8, D), lambda i: (0, 0))
    blk = pl.BlockSpec((MBLK, D), lambda i: (i, 0))
    return pl.pallas_call(
        _k6_body,
        grid=(M // MBLK,),
        in_specs=[blk, st, vec, vec, blk, blk,
                  pl.BlockSpec((MBLK, 1), lambda i: (i, 0))],
        out_specs=blk,
        out_shape=jax.ShapeDtypeStruct((M, D), jnp.float32),
    )(final, s5, g5, b5, mean_feat, xmax, cntpos)


def kernel(inputs, unq_inv, W_lin, b_lin, g_norm, be_norm, W_w1, b_w1, g_w, be_w, W_w2, b_w2, W_pm, b_pm, g_pm, be_pm, W_pw, b_pw, g_lk, be_lk):
    r2 = lambda v: v.reshape(1, -1)
    A, s1 = _k1(inputs, W_lin, r2(b_lin))
    x, B, C, s2, s3 = _k2(A, s1, r2(g_norm), r2(be_norm), W_w1, r2(b_w1), W_pm, r2(b_pm))
    weight = _k3a(B, s2, r2(g_w), r2(be_w), W_w2, b_w2.reshape(1, 1))

    counts = jax.ops.segment_sum(jnp.ones((N,), jnp.float32), unq_inv, num_segments=M)
    wmax = jax.ops.segment_max(weight[:, 0], unq_inv, num_segments=M)
    wmax = jnp.where(counts > 0, wmax, 0.0)
    e = jnp.exp(weight[:, 0] - wmax[unq_inv])
    denom = jax.ops.segment_sum(e, unq_inv, num_segments=M)
    inv_scale = jnp.where(counts > 0, 1.0 / (denom * jnp.clip(counts, 1.0, None)), 0.0)
    cntpos = (counts > 0).astype(jnp.float32)

    fs, fc, ex = _k3b(C, x, e[:, None], s3, r2(g_pm), r2(be_pm), W_pw, b_pw.reshape(1, -1))

    sum_ex = jax.ops.segment_sum(ex, unq_inv, num_segments=M)
    add_s = jax.ops.segment_sum(fs, unq_inv, num_segments=M)
    add_c = jax.ops.segment_sum(fc, unq_inv, num_segments=M)
    xmax = jax.ops.segment_max(x, unq_inv, num_segments=M)

    mean_feat, Dm, s4 = _k4(sum_ex, inv_scale[:, None], W_pm, r2(b_pm))
    final, s5 = _k5(Dm, s4, r2(g_pm), r2(be_pm), W_pw, r2(b_pw), add_s, add_c)
    return _k6(final, s5, r2(g_lk), r2(be_lk), mean_feat, xmax, cntpos[:, None])


# speedup vs baseline: 1.9004x; 1.6711x over previous
import functools

import jax
import jax.numpy as jnp
from jax import lax
from jax.experimental import pallas as pl
from jax.experimental.pallas import tpu as pltpu
from jax.experimental.pallas import tpu_sc as plsc

M = 10000
N = 160000
D = 256
BLK = 1000
MBLK = 1000


def _stats(sums_ref, n, eps, g, b):
    mean = sums_ref[0:1, :] / n
    var = sums_ref[1:2, :] / n - mean * mean
    scale = g * lax.rsqrt(var + eps)
    shift = b - mean * scale
    return scale, shift


def _acc_stats(ref, val):
    i = pl.program_id(0)

    @pl.when(i == 0)
    def _():
        ref[...] = jnp.zeros_like(ref)

    s = jnp.sum(val, axis=0, keepdims=True)
    sq = jnp.sum(val * val, axis=0, keepdims=True)
    ref[0:2, :] += jnp.concatenate([s, sq], axis=0)


def _k1_body(inp_ref, w_ref, b_ref, a_ref, s1_ref):
    a = jnp.dot(inp_ref[...], w_ref[...], preferred_element_type=jnp.float32) + b_ref[...]
    a_ref[...] = a
    _acc_stats(s1_ref, a)


def _k1(inputs, W_lin, b_lin):
    return pl.pallas_call(
        _k1_body,
        grid=(N // BLK,),
        in_specs=[
            pl.BlockSpec((BLK, 128), lambda i: (i, 0)),
            pl.BlockSpec((128, D), lambda i: (0, 0)),
            pl.BlockSpec((1, D), lambda i: (0, 0)),
        ],
        out_specs=[
            pl.BlockSpec((BLK, D), lambda i: (i, 0)),
            pl.BlockSpec((8, D), lambda i: (0, 0)),
        ],
        out_shape=[
            jax.ShapeDtypeStruct((N, D), jnp.float32),
            jax.ShapeDtypeStruct((8, D), jnp.float32),
        ],
    )(inputs, W_lin, b_lin)


def _k2_body(a_ref, s1_ref, g1_ref, b1_ref, ww1_ref, bw1_ref, wpm_ref, bpm_ref,
             x_ref, bb_ref, cc_ref, s2_ref, s3_ref):
    scale, shift = _stats(s1_ref, N, 1e-3, g1_ref[...], b1_ref[...])
    x = jax.nn.relu(a_ref[...] * scale + shift)
    x_ref[...] = x
    bmat = jnp.dot(x, ww1_ref[...], preferred_element_type=jnp.float32) + bw1_ref[...]
    cmat = jnp.dot(x, wpm_ref[...], preferred_element_type=jnp.float32) + bpm_ref[...]
    bb_ref[...] = bmat
    cc_ref[...] = cmat
    _acc_stats(s2_ref, bmat)
    _acc_stats(s3_ref, cmat)


def _k2(A, s1, g1, b1, Ww1, bw1, Wpm, bpm):
    full = pl.BlockSpec((D, D), lambda i: (0, 0))
    vec = pl.BlockSpec((1, D), lambda i: (0, 0))
    st = pl.BlockSpec((8, D), lambda i: (0, 0))
    blk = pl.BlockSpec((BLK, D), lambda i: (i, 0))
    return pl.pallas_call(
        _k2_body,
        grid=(N // BLK,),
        in_specs=[blk, st, vec, vec, full, vec, full, vec],
        out_specs=[blk, blk, blk, st, st],
        out_shape=[
            jax.ShapeDtypeStruct((N, D), jnp.float32),
            jax.ShapeDtypeStruct((N, D), jnp.float32),
            jax.ShapeDtypeStruct((N, D), jnp.float32),
            jax.ShapeDtypeStruct((8, D), jnp.float32),
            jax.ShapeDtypeStruct((8, D), jnp.float32),
        ],
    )(A, s1, g1, b1, Ww1, bw1, Wpm, bpm)


def _k3a_body(bb_ref, s2_ref, g2_ref, b2_ref, ww2_ref, bw2_ref, w_ref):
    scale, shift = _stats(s2_ref, N, 1e-5, g2_ref[...], b2_ref[...])
    h = jax.nn.relu(bb_ref[...] * scale + shift)
    w_ref[...] = jnp.dot(h, ww2_ref[...], preferred_element_type=jnp.float32) + bw2_ref[...]


def _k3a(B, s2, g2, b2, Ww2, bw2):
    vec = pl.BlockSpec((1, D), lambda i: (0, 0))
    st = pl.BlockSpec((8, D), lambda i: (0, 0))
    blk = pl.BlockSpec((BLK, D), lambda i: (i, 0))
    return pl.pallas_call(
        _k3a_body,
        grid=(N // BLK,),
        in_specs=[blk, st, vec, vec,
                  pl.BlockSpec((D, 1), lambda i: (0, 0)),
                  pl.BlockSpec((1, 1), lambda i: (0, 0))],
        out_specs=pl.BlockSpec((BLK, 1), lambda i: (i, 0)),
        out_shape=jax.ShapeDtypeStruct((N, 1), jnp.float32),
    )(B, s2, g2, b2, Ww2, bw2)


def _k3b_body(c_ref, x_ref, e_ref, s3_ref, g3_ref, b3_ref, wpw_ref, bpw_ref,
              fs_ref, fc_ref, ex_ref):
    scale, shift = _stats(s3_ref, N, 1e-3, g3_ref[...], b3_ref[...])
    pxyz = c_ref[...] * scale + shift
    pw = jnp.dot(pxyz, wpw_ref[...], preferred_element_type=jnp.float32) + bpw_ref[...]
    fs_ref[...] = pxyz * jnp.sin(pw)
    fc_ref[...] = pxyz * jnp.cos(pw)
    ex_ref[...] = x_ref[...] * e_ref[...]


def _k3b(C, x, e, s3, g3, b3, Wpw, bpw):
    vec = pl.BlockSpec((1, D), lambda i: (0, 0))
    st = pl.BlockSpec((8, D), lambda i: (0, 0))
    blk = pl.BlockSpec((BLK, D), lambda i: (i, 0))
    return pl.pallas_call(
        _k3b_body,
        grid=(N // BLK,),
        in_specs=[blk, blk, pl.BlockSpec((BLK, 1), lambda i: (i, 0)), st, vec, vec,
                  pl.BlockSpec((D, D), lambda i: (0, 0)), vec],
        out_specs=[blk, blk, blk],
        out_shape=[jax.ShapeDtypeStruct((N, D), jnp.float32)] * 3,
    )(C, x, e, s3, g3, b3, Wpw, bpw)


def _k4_body(se_ref, inv_ref, wpm_ref, bpm_ref, mf_ref, d_ref, s4_ref):
    mf = se_ref[...] * inv_ref[...]
    mf_ref[...] = mf
    d = jnp.dot(mf, wpm_ref[...], preferred_element_type=jnp.float32) + bpm_ref[...]
    d_ref[...] = d
    _acc_stats(s4_ref, d)


def _k4(sum_ex, inv_scale, Wpm, bpm):
    vec = pl.BlockSpec((1, D), lambda i: (0, 0))
    st = pl.BlockSpec((8, D), lambda i: (0, 0))
    blk = pl.BlockSpec((MBLK, D), lambda i: (i, 0))
    return pl.pallas_call(
        _k4_body,
        grid=(M // MBLK,),
        in_specs=[blk, pl.BlockSpec((MBLK, 1), lambda i: (i, 0)),
                  pl.BlockSpec((D, D), lambda i: (0, 0)), vec],
        out_specs=[blk, blk, st],
        out_shape=[
            jax.ShapeDtypeStruct((M, D), jnp.float32),
            jax.ShapeDtypeStruct((M, D), jnp.float32),
            jax.ShapeDtypeStruct((8, D), jnp.float32),
        ],
    )(sum_ex, inv_scale, Wpm, bpm)


def _k5_body(d_ref, s4_ref, g3_ref, b3_ref, wpw_ref, bpw_ref, as_ref, ac_ref,
             fin_ref, s5_ref):
    scale, shift = _stats(s4_ref, M, 1e-3, g3_ref[...], b3_ref[...])
    feat = d_ref[...] * scale + shift
    mw = jnp.dot(feat, wpw_ref[...], preferred_element_type=jnp.float32) + bpw_ref[...]
    cs = feat * jnp.sin(mw)
    cc = feat * jnp.cos(mw)
    final = (as_ref[...] + cs) * cs + (ac_ref[...] + cc) * cc
    fin_ref[...] = final
    _acc_stats(s5_ref, final)


def _k5(Dm, s4, g3, b3, Wpw, bpw, add_s, add_c):
    vec = pl.BlockSpec((1, D), lambda i: (0, 0))
    st = pl.BlockSpec((8, D), lambda i: (0, 0))
    blk = pl.BlockSpec((MBLK, D), lambda i: (i, 0))
    return pl.pallas_call(
        _k5_body,
        grid=(M // MBLK,),
        in_specs=[blk, st, vec, vec, pl.BlockSpec((D, D), lambda i: (0, 0)), vec,
                  blk, blk],
        out_specs=[blk, st],
        out_shape=[
            jax.ShapeDtypeStruct((M, D), jnp.float32),
            jax.ShapeDtypeStruct((8, D), jnp.float32),
        ],
    )(Dm, s4, g3, b3, Wpw, bpw, add_s, add_c)


def _k6_body(fin_ref, s5_ref, g5_ref, b5_ref, mf_ref, xm_ref, cp_ref, out_ref):
    scale, shift = _stats(s5_ref, M, 1e-3, g5_ref[...], b5_ref[...])
    link = jax.nn.relu(fin_ref[...] * scale + shift)
    xmax = jnp.where(cp_ref[...] > 0, xm_ref[...], 0.0)
    out_ref[...] = ((mf_ref[...] + link) * 0.5 + xmax) * 0.5


def _k6(final, s5, g5, b5, mean_feat, xmax, cntpos):
    vec = pl.BlockSpec((1, D), lambda i: (0, 0))
    st = pl.BlockSpec((8, D), lambda i: (0, 0))
    blk = pl.BlockSpec((MBLK, D), lambda i: (i, 0))
    return pl.pallas_call(
        _k6_body,
        grid=(M // MBLK,),
        in_specs=[blk, st, vec, vec, blk, blk,
                  pl.BlockSpec((MBLK, 1), lambda i: (i, 0))],
        out_specs=blk,
        out_shape=jax.ShapeDtypeStruct((M, D), jnp.float32),
    )(final, s5, g5, b5, mean_feat, xmax, cntpos)


MPAD = ((M + 127) // 128) * 128


def _s3_body(ids_ref, a0_ref, a1_ref, a2_ref, o0_ref, o1_ref, o2_ref):
    i = pl.program_id(0)

    @pl.when(i == 0)
    def _():
        o0_ref[...] = jnp.zeros_like(o0_ref)
        o1_ref[...] = jnp.zeros_like(o1_ref)
        o2_ref[...] = jnp.zeros_like(o2_ref)

    ids = ids_ref[...]
    w_lo = ids_ref[0, 0] // 128
    w_hi = ids_ref[BLK - 1, 0] // 128
    cols = jax.lax.broadcasted_iota(jnp.int32, (1, 128), 1)

    def win(w, _):
        onehot = (ids == w * 128 + cols).astype(jnp.float32)
        dn = (((0,), (0,)), ((), ()))
        for a_ref, o_ref in ((a0_ref, o0_ref), (a1_ref, o1_ref), (a2_ref, o2_ref)):
            psum = lax.dot_general(onehot, a_ref[...], dn,
                                   preferred_element_type=jnp.float32)
            o_ref[pl.ds(w * 128, 128), :] += psum
        return None

    lax.fori_loop(w_lo, w_hi + 1, win, None)


def _s3(ids, ex, fs, fc):
    blk = pl.BlockSpec((BLK, D), lambda i: (i, 0))
    out = pl.BlockSpec((MPAD, D), lambda i: (0, 0))
    res = pl.pallas_call(
        _s3_body,
        grid=(N // BLK,),
        in_specs=[pl.BlockSpec((BLK, 1), lambda i: (i, 0)), blk, blk, blk],
        out_specs=[out, out, out],
        out_shape=[jax.ShapeDtypeStruct((MPAD, D), jnp.float32)] * 3,
    )(ids, ex, fs, fc)
    return tuple(r[:M] for r in res)


def kernel(inputs, unq_inv, W_lin, b_lin, g_norm, be_norm, W_w1, b_w1, g_w, be_w, W_w2, b_w2, W_pm, b_pm, g_pm, be_pm, W_pw, b_pw, g_lk, be_lk):
    r2 = lambda v: v.reshape(1, -1)
    A, s1 = _k1(inputs, W_lin, r2(b_lin))
    x, B, C, s2, s3 = _k2(A, s1, r2(g_norm), r2(be_norm), W_w1, r2(b_w1), W_pm, r2(b_pm))
    weight = _k3a(B, s2, r2(g_w), r2(be_w), W_w2, b_w2.reshape(1, 1))

    counts = jax.ops.segment_sum(jnp.ones((N,), jnp.float32), unq_inv, num_segments=M)
    wmax = jax.ops.segment_max(weight[:, 0], unq_inv, num_segments=M)
    wmax = jnp.where(counts > 0, wmax, 0.0)
    e = jnp.exp(weight[:, 0] - wmax[unq_inv])
    denom = jax.ops.segment_sum(e, unq_inv, num_segments=M)
    inv_scale = jnp.where(counts > 0, 1.0 / (denom * jnp.clip(counts, 1.0, None)), 0.0)
    cntpos = (counts > 0).astype(jnp.float32)

    fs, fc, ex = _k3b(C, x, e[:, None], s3, r2(g_pm), r2(be_pm), W_pw, b_pw.reshape(1, -1))

    sum_ex, add_s, add_c = _s3(unq_inv.astype(jnp.int32).reshape(-1, 1), ex, fs, fc)
    xmax = jax.ops.segment_max(x, unq_inv, num_segments=M)

    mean_feat, Dm, s4 = _k4(sum_ex, inv_scale[:, None], W_pm, r2(b_pm))
    final, s5 = _k5(Dm, s4, r2(g_pm), r2(be_pm), W_pw, r2(b_pw), add_s, add_c)
    return _k6(final, s5, r2(g_lk), r2(be_lk), mean_feat, xmax, cntpos[:, None])


# speedup vs baseline: 2.6415x; 1.3900x over previous
import functools

import jax
import jax.numpy as jnp
from jax import lax
from jax.experimental import pallas as pl
from jax.experimental.pallas import tpu as pltpu
from jax.experimental.pallas import tpu_sc as plsc

M = 10000
N = 160000
D = 256
BLK = 1000
MBLK = 1000


def _stats(sums_ref, n, eps, g, b):
    mean = sums_ref[0:1, :] / n
    var = sums_ref[1:2, :] / n - mean * mean
    scale = g * lax.rsqrt(var + eps)
    shift = b - mean * scale
    return scale, shift


def _acc_stats(ref, val):
    i = pl.program_id(0)

    @pl.when(i == 0)
    def _():
        ref[...] = jnp.zeros_like(ref)

    s = jnp.sum(val, axis=0, keepdims=True)
    sq = jnp.sum(val * val, axis=0, keepdims=True)
    ref[0:2, :] += jnp.concatenate([s, sq], axis=0)


def _k1_body(inp_ref, w_ref, b_ref, a_ref, s1_ref):
    a = jnp.dot(inp_ref[...], w_ref[...], preferred_element_type=jnp.float32) + b_ref[...]
    a_ref[...] = a
    _acc_stats(s1_ref, a)


def _k1(inputs, W_lin, b_lin):
    return pl.pallas_call(
        _k1_body,
        grid=(N // BLK,),
        in_specs=[
            pl.BlockSpec((BLK, 128), lambda i: (i, 0)),
            pl.BlockSpec((128, D), lambda i: (0, 0)),
            pl.BlockSpec((1, D), lambda i: (0, 0)),
        ],
        out_specs=[
            pl.BlockSpec((BLK, D), lambda i: (i, 0)),
            pl.BlockSpec((8, D), lambda i: (0, 0)),
        ],
        out_shape=[
            jax.ShapeDtypeStruct((N, D), jnp.float32),
            jax.ShapeDtypeStruct((8, D), jnp.float32),
        ],
    )(inputs, W_lin, b_lin)


def _k2_body(a_ref, s1_ref, g1_ref, b1_ref, ww1_ref, bw1_ref, wpm_ref, bpm_ref,
             x_ref, bb_ref, cc_ref, s2_ref, s3_ref):
    scale, shift = _stats(s1_ref, N, 1e-3, g1_ref[...], b1_ref[...])
    x = jax.nn.relu(a_ref[...] * scale + shift)
    x_ref[...] = x
    bmat = jnp.dot(x, ww1_ref[...], preferred_element_type=jnp.float32) + bw1_ref[...]
    cmat = jnp.dot(x, wpm_ref[...], preferred_element_type=jnp.float32) + bpm_ref[...]
    bb_ref[...] = bmat
    cc_ref[...] = cmat
    _acc_stats(s2_ref, bmat)
    _acc_stats(s3_ref, cmat)


def _k2(A, s1, g1, b1, Ww1, bw1, Wpm, bpm):
    full = pl.BlockSpec((D, D), lambda i: (0, 0))
    vec = pl.BlockSpec((1, D), lambda i: (0, 0))
    st = pl.BlockSpec((8, D), lambda i: (0, 0))
    blk = pl.BlockSpec((BLK, D), lambda i: (i, 0))
    return pl.pallas_call(
        _k2_body,
        grid=(N // BLK,),
        in_specs=[blk, st, vec, vec, full, vec, full, vec],
        out_specs=[blk, blk, blk, st, st],
        out_shape=[
            jax.ShapeDtypeStruct((N, D), jnp.float32),
            jax.ShapeDtypeStruct((N, D), jnp.float32),
            jax.ShapeDtypeStruct((N, D), jnp.float32),
            jax.ShapeDtypeStruct((8, D), jnp.float32),
            jax.ShapeDtypeStruct((8, D), jnp.float32),
        ],
    )(A, s1, g1, b1, Ww1, bw1, Wpm, bpm)


def _k3a_body(bb_ref, s2_ref, g2_ref, b2_ref, ww2_ref, bw2_ref, w_ref):
    scale, shift = _stats(s2_ref, N, 1e-5, g2_ref[...], b2_ref[...])
    h = jax.nn.relu(bb_ref[...] * scale + shift)
    w_ref[...] = jnp.dot(h, ww2_ref[...], preferred_element_type=jnp.float32) + bw2_ref[...]


def _k3a(B, s2, g2, b2, Ww2, bw2):
    vec = pl.BlockSpec((1, D), lambda i: (0, 0))
    st = pl.BlockSpec((8, D), lambda i: (0, 0))
    blk = pl.BlockSpec((BLK, D), lambda i: (i, 0))
    return pl.pallas_call(
        _k3a_body,
        grid=(N // BLK,),
        in_specs=[blk, st, vec, vec,
                  pl.BlockSpec((D, 1), lambda i: (0, 0)),
                  pl.BlockSpec((1, 1), lambda i: (0, 0))],
        out_specs=pl.BlockSpec((BLK, 1), lambda i: (i, 0)),
        out_shape=jax.ShapeDtypeStruct((N, 1), jnp.float32),
    )(B, s2, g2, b2, Ww2, bw2)


def _k3b_body(c_ref, x_ref, e_ref, s3_ref, g3_ref, b3_ref, wpw_ref, bpw_ref,
              fs_ref, fc_ref, ex_ref):
    scale, shift = _stats(s3_ref, N, 1e-3, g3_ref[...], b3_ref[...])
    pxyz = c_ref[...] * scale + shift
    pw = jnp.dot(pxyz, wpw_ref[...], preferred_element_type=jnp.float32) + bpw_ref[...]
    fs_ref[...] = pxyz * jnp.sin(pw)
    fc_ref[...] = pxyz * jnp.cos(pw)
    ex_ref[...] = x_ref[...] * e_ref[...]


def _k3b(C, x, e, s3, g3, b3, Wpw, bpw):
    vec = pl.BlockSpec((1, D), lambda i: (0, 0))
    st = pl.BlockSpec((8, D), lambda i: (0, 0))
    blk = pl.BlockSpec((BLK, D), lambda i: (i, 0))
    return pl.pallas_call(
        _k3b_body,
        grid=(N // BLK,),
        in_specs=[blk, blk, pl.BlockSpec((BLK, 1), lambda i: (i, 0)), st, vec, vec,
                  pl.BlockSpec((D, D), lambda i: (0, 0)), vec],
        out_specs=[blk, blk, blk],
        out_shape=[jax.ShapeDtypeStruct((N, D), jnp.float32)] * 3,
    )(C, x, e, s3, g3, b3, Wpw, bpw)


def _k4_body(se_ref, inv_ref, wpm_ref, bpm_ref, mf_ref, d_ref, s4_ref):
    mf = se_ref[...] * inv_ref[...]
    mf_ref[...] = mf
    d = jnp.dot(mf, wpm_ref[...], preferred_element_type=jnp.float32) + bpm_ref[...]
    d_ref[...] = d
    _acc_stats(s4_ref, d)


def _k4(sum_ex, inv_scale, Wpm, bpm):
    vec = pl.BlockSpec((1, D), lambda i: (0, 0))
    st = pl.BlockSpec((8, D), lambda i: (0, 0))
    blk = pl.BlockSpec((MBLK, D), lambda i: (i, 0))
    return pl.pallas_call(
        _k4_body,
        grid=(M // MBLK,),
        in_specs=[blk, pl.BlockSpec((MBLK, 1), lambda i: (i, 0)),
                  pl.BlockSpec((D, D), lambda i: (0, 0)), vec],
        out_specs=[blk, blk, st],
        out_shape=[
            jax.ShapeDtypeStruct((M, D), jnp.float32),
            jax.ShapeDtypeStruct((M, D), jnp.float32),
            jax.ShapeDtypeStruct((8, D), jnp.float32),
        ],
    )(sum_ex, inv_scale, Wpm, bpm)


def _k5_body(d_ref, s4_ref, g3_ref, b3_ref, wpw_ref, bpw_ref, as_ref, ac_ref,
             fin_ref, s5_ref):
    scale, shift = _stats(s4_ref, M, 1e-3, g3_ref[...], b3_ref[...])
    feat = d_ref[...] * scale + shift
    mw = jnp.dot(feat, wpw_ref[...], preferred_element_type=jnp.float32) + bpw_ref[...]
    cs = feat * jnp.sin(mw)
    cc = feat * jnp.cos(mw)
    final = (as_ref[...] + cs) * cs + (ac_ref[...] + cc) * cc
    fin_ref[...] = final
    _acc_stats(s5_ref, final)


def _k5(Dm, s4, g3, b3, Wpw, bpw, add_s, add_c):
    vec = pl.BlockSpec((1, D), lambda i: (0, 0))
    st = pl.BlockSpec((8, D), lambda i: (0, 0))
    blk = pl.BlockSpec((MBLK, D), lambda i: (i, 0))
    return pl.pallas_call(
        _k5_body,
        grid=(M // MBLK,),
        in_specs=[blk, st, vec, vec, pl.BlockSpec((D, D), lambda i: (0, 0)), vec,
                  blk, blk],
        out_specs=[blk, st],
        out_shape=[
            jax.ShapeDtypeStruct((M, D), jnp.float32),
            jax.ShapeDtypeStruct((8, D), jnp.float32),
        ],
    )(Dm, s4, g3, b3, Wpw, bpw, add_s, add_c)


def _k6_body(fin_ref, s5_ref, g5_ref, b5_ref, mf_ref, xm_ref, cp_ref, out_ref):
    scale, shift = _stats(s5_ref, M, 1e-3, g5_ref[...], b5_ref[...])
    link = jax.nn.relu(fin_ref[...] * scale + shift)
    xmax = jnp.where(cp_ref[...] > 0, xm_ref[...], 0.0)
    out_ref[...] = ((mf_ref[...] + link) * 0.5 + xmax) * 0.5


def _k6(final, s5, g5, b5, mean_feat, xmax, cntpos):
    vec = pl.BlockSpec((1, D), lambda i: (0, 0))
    st = pl.BlockSpec((8, D), lambda i: (0, 0))
    blk = pl.BlockSpec((MBLK, D), lambda i: (i, 0))
    return pl.pallas_call(
        _k6_body,
        grid=(M // MBLK,),
        in_specs=[blk, st, vec, vec, blk, blk,
                  pl.BlockSpec((MBLK, 1), lambda i: (i, 0))],
        out_specs=blk,
        out_shape=jax.ShapeDtypeStruct((M, D), jnp.float32),
    )(final, s5, g5, b5, mean_feat, xmax, cntpos)


MPAD = ((M + 127) // 128) * 128


def _s3_body(ids_ref, a0_ref, a1_ref, a2_ref, o0_ref, o1_ref, o2_ref):
    i = pl.program_id(0)

    @pl.when(i == 0)
    def _():
        o0_ref[...] = jnp.zeros_like(o0_ref)
        o1_ref[...] = jnp.zeros_like(o1_ref)
        o2_ref[...] = jnp.zeros_like(o2_ref)

    ids = ids_ref[...]
    w_lo = ids_ref[0, 0] // 128
    w_hi = ids_ref[BLK - 1, 0] // 128
    cols = jax.lax.broadcasted_iota(jnp.int32, (1, 128), 1)

    def win(w, _):
        onehot = (ids == w * 128 + cols).astype(jnp.float32)
        dn = (((0,), (0,)), ((), ()))
        for a_ref, o_ref in ((a0_ref, o0_ref), (a1_ref, o1_ref), (a2_ref, o2_ref)):
            psum = lax.dot_general(onehot, a_ref[...], dn,
                                   preferred_element_type=jnp.float32)
            o_ref[pl.ds(w * 128, 128), :] += psum
        return None

    lax.fori_loop(w_lo, w_hi + 1, win, None)


def _s3(ids, ex, fs, fc):
    blk = pl.BlockSpec((BLK, D), lambda i: (i, 0))
    out = pl.BlockSpec((MPAD, D), lambda i: (0, 0))
    res = pl.pallas_call(
        _s3_body,
        grid=(N // BLK,),
        in_specs=[pl.BlockSpec((BLK, 1), lambda i: (i, 0)), blk, blk, blk],
        out_specs=[out, out, out],
        out_shape=[jax.ShapeDtypeStruct((MPAD, D), jnp.float32)] * 3,
    )(ids, ex, fs, fc)
    return tuple(r[:M] for r in res)


WPAD = MPAD // 128


def _p1_body(ids_ref, w_ref, cnt_ref, wm_ref):
    i = pl.program_id(0)

    @pl.when(i == 0)
    def _():
        cnt_ref[...] = jnp.zeros_like(cnt_ref)
        wm_ref[...] = jnp.full_like(wm_ref, -3.4e38)

    ids = ids_ref[...]
    wv = w_ref[...]
    w_lo = ids_ref[0, 0] // 128
    w_hi = ids_ref[BLK - 1, 0] // 128
    cols = lax.broadcasted_iota(jnp.int32, (1, 128), 1)

    def win(w, _):
        oh = ids == w * 128 + cols
        cnt_ref[pl.ds(w, 1), :] += jnp.sum(oh.astype(jnp.float32), axis=0,
                                           keepdims=True)
        vals = jnp.where(oh, wv, -3.4e38)
        wm_ref[pl.ds(w, 1), :] = jnp.maximum(
            wm_ref[pl.ds(w, 1), :], jnp.max(vals, axis=0, keepdims=True))
        return None

    lax.fori_loop(w_lo, w_hi + 1, win, None)


def _p1(ids, weight):
    out = pl.BlockSpec((WPAD, 128), lambda i: (0, 0))
    return pl.pallas_call(
        _p1_body,
        grid=(N // BLK,),
        in_specs=[pl.BlockSpec((BLK, 1), lambda i: (i, 0))] * 2,
        out_specs=[out, out],
        out_shape=[jax.ShapeDtypeStruct((WPAD, 128), jnp.float32)] * 2,
    )(ids, weight)


def _p2_body(ids_ref, w_ref, wm_ref, e_ref, den_ref):
    i = pl.program_id(0)

    @pl.when(i == 0)
    def _():
        den_ref[...] = jnp.zeros_like(den_ref)

    ids = ids_ref[...]
    wv = w_ref[...]
    w_lo = ids_ref[0, 0] // 128
    w_hi = ids_ref[BLK - 1, 0] // 128
    cols = lax.broadcasted_iota(jnp.int32, (1, 128), 1)

    def win1(w, g):
        oh = (ids == w * 128 + cols).astype(jnp.float32)
        return g + jnp.sum(oh * wm_ref[pl.ds(w, 1), :], axis=1, keepdims=True)

    gath = lax.fori_loop(w_lo, w_hi + 1, win1,
                         jnp.zeros((BLK, 1), jnp.float32))
    e = jnp.exp(wv - gath)
    e_ref[...] = e

    def win2(w, _):
        oh = (ids == w * 128 + cols).astype(jnp.float32)
        den_ref[pl.ds(w, 1), :] += jnp.sum(oh * e, axis=0, keepdims=True)
        return None

    lax.fori_loop(w_lo, w_hi + 1, win2, None)


def _p2(ids, weight, wm):
    return pl.pallas_call(
        _p2_body,
        grid=(N // BLK,),
        in_specs=[pl.BlockSpec((BLK, 1), lambda i: (i, 0))] * 2 +
                 [pl.BlockSpec((WPAD, 128), lambda i: (0, 0))],
        out_specs=[pl.BlockSpec((BLK, 1), lambda i: (i, 0)),
                   pl.BlockSpec((WPAD, 128), lambda i: (0, 0))],
        out_shape=[jax.ShapeDtypeStruct((N, 1), jnp.float32),
                   jax.ShapeDtypeStruct((WPAD, 128), jnp.float32)],
    )(ids, weight, wm)


def kernel(inputs, unq_inv, W_lin, b_lin, g_norm, be_norm, W_w1, b_w1, g_w, be_w, W_w2, b_w2, W_pm, b_pm, g_pm, be_pm, W_pw, b_pw, g_lk, be_lk):
    r2 = lambda v: v.reshape(1, -1)
    A, s1 = _k1(inputs, W_lin, r2(b_lin))
    x, B, C, s2, s3 = _k2(A, s1, r2(g_norm), r2(be_norm), W_w1, r2(b_w1), W_pm, r2(b_pm))
    weight = _k3a(B, s2, r2(g_w), r2(be_w), W_w2, b_w2.reshape(1, 1))

    ids2 = unq_inv.astype(jnp.int32).reshape(-1, 1)
    cntw, wmw = _p1(ids2, weight)
    e2, denw = _p2(ids2, weight, wmw)
    counts = cntw.reshape(-1)[:M]
    denom = denw.reshape(-1)[:M]
    inv_scale = jnp.where(counts > 0, 1.0 / (denom * jnp.clip(counts, 1.0, None)), 0.0)
    cntpos = (counts > 0).astype(jnp.float32)

    fs, fc, ex = _k3b(C, x, e2, s3, r2(g_pm), r2(be_pm), W_pw, b_pw.reshape(1, -1))

    sum_ex, add_s, add_c = _s3(ids2, ex, fs, fc)
    xmax = jax.ops.segment_max(x, unq_inv, num_segments=M)

    mean_feat, Dm, s4 = _k4(sum_ex, inv_scale[:, None], W_pm, r2(b_pm))
    final, s5 = _k5(Dm, s4, r2(g_pm), r2(be_pm), W_pw, r2(b_pw), add_s, add_c)
    return _k6(final, s5, r2(g_lk), r2(be_lk), mean_feat, xmax, cntpos[:, None])


# speedup vs baseline: 2.6435x; 1.0008x over previous
import functools

import jax
import jax.numpy as jnp
from jax import lax
from jax.experimental import pallas as pl
from jax.experimental.pallas import tpu as pltpu

M = 10000
N = 160000
D = 256
BLK = 1000
MBLK = 1000


def _stats(sums_ref, n, eps, g, b):
    mean = sums_ref[0:1, :] / n
    var = sums_ref[1:2, :] / n - mean * mean
    scale = g * lax.rsqrt(var + eps)
    shift = b - mean * scale
    return scale, shift


def _acc_stats(ref, val):
    i = pl.program_id(0)

    @pl.when(i == 0)
    def _():
        ref[...] = jnp.zeros_like(ref)

    s = jnp.sum(val, axis=0, keepdims=True)
    sq = jnp.sum(val * val, axis=0, keepdims=True)
    ref[0:2, :] += jnp.concatenate([s, sq], axis=0)


def _k1_body(inp_ref, w_ref, b_ref, a_ref, s1_ref):
    a = jnp.dot(inp_ref[...], w_ref[...], preferred_element_type=jnp.float32) + b_ref[...]
    a_ref[...] = a
    _acc_stats(s1_ref, a)


def _k1(inputs, W_lin, b_lin):
    return pl.pallas_call(
        _k1_body,
        grid=(N // BLK,),
        in_specs=[
            pl.BlockSpec((BLK, 128), lambda i: (i, 0)),
            pl.BlockSpec((128, D), lambda i: (0, 0)),
            pl.BlockSpec((1, D), lambda i: (0, 0)),
        ],
        out_specs=[
            pl.BlockSpec((BLK, D), lambda i: (i, 0)),
            pl.BlockSpec((8, D), lambda i: (0, 0)),
        ],
        out_shape=[
            jax.ShapeDtypeStruct((N, D), jnp.float32),
            jax.ShapeDtypeStruct((8, D), jnp.float32),
        ],
    )(inputs, W_lin, b_lin)


def _k2_body(a_ref, s1_ref, g1_ref, b1_ref, ww1_ref, bw1_ref, wpm_ref, bpm_ref,
             x_ref, bb_ref, cc_ref, s2_ref, s3_ref):
    scale, shift = _stats(s1_ref, N, 1e-3, g1_ref[...], b1_ref[...])
    x = jax.nn.relu(a_ref[...] * scale + shift)
    x_ref[...] = x
    bmat = jnp.dot(x, ww1_ref[...], preferred_element_type=jnp.float32) + bw1_ref[...]
    cmat = jnp.dot(x, wpm_ref[...], preferred_element_type=jnp.float32) + bpm_ref[...]
    bb_ref[...] = bmat
    cc_ref[...] = cmat
    _acc_stats(s2_ref, bmat)
    _acc_stats(s3_ref, cmat)


def _k2(A, s1, g1, b1, Ww1, bw1, Wpm, bpm):
    full = pl.BlockSpec((D, D), lambda i: (0, 0))
    vec = pl.BlockSpec((1, D), lambda i: (0, 0))
    st = pl.BlockSpec((8, D), lambda i: (0, 0))
    blk = pl.BlockSpec((BLK, D), lambda i: (i, 0))
    return pl.pallas_call(
        _k2_body,
        grid=(N // BLK,),
        in_specs=[blk, st, vec, vec, full, vec, full, vec],
        out_specs=[blk, blk, blk, st, st],
        out_shape=[
            jax.ShapeDtypeStruct((N, D), jnp.float32),
            jax.ShapeDtypeStruct((N, D), jnp.float32),
            jax.ShapeDtypeStruct((N, D), jnp.float32),
            jax.ShapeDtypeStruct((8, D), jnp.float32),
            jax.ShapeDtypeStruct((8, D), jnp.float32),
        ],
    )(A, s1, g1, b1, Ww1, bw1, Wpm, bpm)


def _k3a_body(bb_ref, s2_ref, g2_ref, b2_ref, ww2_ref, bw2_ref, w_ref):
    scale, shift = _stats(s2_ref, N, 1e-5, g2_ref[...], b2_ref[...])
    h = jax.nn.relu(bb_ref[...] * scale + shift)
    w_ref[...] = jnp.dot(h, ww2_ref[...], preferred_element_type=jnp.float32) + bw2_ref[...]


def _k3a(B, s2, g2, b2, Ww2, bw2):
    vec = pl.BlockSpec((1, D), lambda i: (0, 0))
    st = pl.BlockSpec((8, D), lambda i: (0, 0))
    blk = pl.BlockSpec((BLK, D), lambda i: (i, 0))
    return pl.pallas_call(
        _k3a_body,
        grid=(N // BLK,),
        in_specs=[blk, st, vec, vec,
                  pl.BlockSpec((D, 1), lambda i: (0, 0)),
                  pl.BlockSpec((1, 1), lambda i: (0, 0))],
        out_specs=pl.BlockSpec((BLK, 1), lambda i: (i, 0)),
        out_shape=jax.ShapeDtypeStruct((N, 1), jnp.float32),
    )(B, s2, g2, b2, Ww2, bw2)


def _k3b_body(c_ref, x_ref, e_ref, s3_ref, g3_ref, b3_ref, wpw_ref, bpw_ref,
              fs_ref, fc_ref, ex_ref):
    scale, shift = _stats(s3_ref, N, 1e-3, g3_ref[...], b3_ref[...])
    pxyz = c_ref[...] * scale + shift
    pw = jnp.dot(pxyz, wpw_ref[...], preferred_element_type=jnp.float32) + bpw_ref[...]
    fs_ref[...] = pxyz * jnp.sin(pw)
    fc_ref[...] = pxyz * jnp.cos(pw)
    ex_ref[...] = x_ref[...] * e_ref[...]


def _k3b(C, x, e, s3, g3, b3, Wpw, bpw):
    vec = pl.BlockSpec((1, D), lambda i: (0, 0))
    st = pl.BlockSpec((8, D), lambda i: (0, 0))
    blk = pl.BlockSpec((BLK, D), lambda i: (i, 0))
    return pl.pallas_call(
        _k3b_body,
        grid=(N // BLK,),
        in_specs=[blk, blk, pl.BlockSpec((BLK, 1), lambda i: (i, 0)), st, vec, vec,
                  pl.BlockSpec((D, D), lambda i: (0, 0)), vec],
        out_specs=[blk, blk, blk],
        out_shape=[jax.ShapeDtypeStruct((N, D), jnp.float32)] * 3,
    )(C, x, e, s3, g3, b3, Wpw, bpw)


def _k4_body(se_ref, inv_ref, wpm_ref, bpm_ref, mf_ref, d_ref, s4_ref):
    mf = se_ref[...] * inv_ref[...]
    mf_ref[...] = mf
    d = jnp.dot(mf, wpm_ref[...], preferred_element_type=jnp.float32) + bpm_ref[...]
    d_ref[...] = d
    _acc_stats(s4_ref, d)


def _k4(sum_ex, inv_scale, Wpm, bpm):
    vec = pl.BlockSpec((1, D), lambda i: (0, 0))
    st = pl.BlockSpec((8, D), lambda i: (0, 0))
    blk = pl.BlockSpec((MBLK, D), lambda i: (i, 0))
    return pl.pallas_call(
        _k4_body,
        grid=(M // MBLK,),
        in_specs=[blk, pl.BlockSpec((MBLK, 1), lambda i: (i, 0)),
                  pl.BlockSpec((D, D), lambda i: (0, 0)), vec],
        out_specs=[blk, blk, st],
        out_shape=[
            jax.ShapeDtypeStruct((M, D), jnp.float32),
            jax.ShapeDtypeStruct((M, D), jnp.float32),
            jax.ShapeDtypeStruct((8, D), jnp.float32),
        ],
    )(sum_ex, inv_scale, Wpm, bpm)


def _k5_body(d_ref, s4_ref, g3_ref, b3_ref, wpw_ref, bpw_ref, as_ref, ac_ref,
             fin_ref, s5_ref):
    scale, shift = _stats(s4_ref, M, 1e-3, g3_ref[...], b3_ref[...])
    feat = d_ref[...] * scale + shift
    mw = jnp.dot(feat, wpw_ref[...], preferred_element_type=jnp.float32) + bpw_ref[...]
    cs = feat * jnp.sin(mw)
    cc = feat * jnp.cos(mw)
    final = (as_ref[...] + cs) * cs + (ac_ref[...] + cc) * cc
    fin_ref[...] = final
    _acc_stats(s5_ref, final)


def _k5(Dm, s4, g3, b3, Wpw, bpw, add_s, add_c):
    vec = pl.BlockSpec((1, D), lambda i: (0, 0))
    st = pl.BlockSpec((8, D), lambda i: (0, 0))
    blk = pl.BlockSpec((MBLK, D), lambda i: (i, 0))
    return pl.pallas_call(
        _k5_body,
        grid=(M // MBLK,),
        in_specs=[blk, st, vec, vec, pl.BlockSpec((D, D), lambda i: (0, 0)), vec,
                  blk, blk],
        out_specs=[blk, st],
        out_shape=[
            jax.ShapeDtypeStruct((M, D), jnp.float32),
            jax.ShapeDtypeStruct((8, D), jnp.float32),
        ],
    )(Dm, s4, g3, b3, Wpw, bpw, add_s, add_c)


def _k6_body(fin_ref, s5_ref, g5_ref, b5_ref, mf_ref, xm_ref, cp_ref, out_ref):
    scale, shift = _stats(s5_ref, M, 1e-3, g5_ref[...], b5_ref[...])
    link = jax.nn.relu(fin_ref[...] * scale + shift)
    xmax = jnp.where(cp_ref[...] > 0, xm_ref[...], 0.0)
    out_ref[...] = ((mf_ref[...] + link) * 0.5 + xmax) * 0.5


def _k6(final, s5, g5, b5, mean_feat, xmax, cntpos):
    vec = pl.BlockSpec((1, D), lambda i: (0, 0))
    st = pl.BlockSpec((8, D), lambda i: (0, 0))
    blk = pl.BlockSpec((MBLK, D), lambda i: (i, 0))
    return pl.pallas_call(
        _k6_body,
        grid=(M // MBLK,),
        in_specs=[blk, st, vec, vec, blk, blk,
                  pl.BlockSpec((MBLK, 1), lambda i: (i, 0))],
        out_specs=blk,
        out_shape=jax.ShapeDtypeStruct((M, D), jnp.float32),
    )(final, s5, g5, b5, mean_feat, xmax, cntpos)


MPAD = ((M + 127) // 128) * 128


def _s3_body(ids_ref, a0_ref, a1_ref, a2_ref, o0_ref, o1_ref, o2_ref):
    i = pl.program_id(0)

    @pl.when(i == 0)
    def _():
        o0_ref[...] = jnp.zeros_like(o0_ref)
        o1_ref[...] = jnp.zeros_like(o1_ref)
        o2_ref[...] = jnp.zeros_like(o2_ref)

    ids = ids_ref[...]
    w_lo = ids_ref[0, 0] // 128
    w_hi = ids_ref[BLK - 1, 0] // 128
    cols = jax.lax.broadcasted_iota(jnp.int32, (1, 128), 1)

    def win(w, _):
        onehot = (ids == w * 128 + cols).astype(jnp.float32)
        dn = (((0,), (0,)), ((), ()))
        for a_ref, o_ref in ((a0_ref, o0_ref), (a1_ref, o1_ref), (a2_ref, o2_ref)):
            psum = lax.dot_general(onehot, a_ref[...], dn,
                                   preferred_element_type=jnp.float32)
            o_ref[pl.ds(w * 128, 128), :] += psum
        return None

    lax.fori_loop(w_lo, w_hi + 1, win, None)


def _s3(ids, ex, fs, fc):
    blk = pl.BlockSpec((BLK, D), lambda i: (i, 0))
    out = pl.BlockSpec((MPAD, D), lambda i: (0, 0))
    res = pl.pallas_call(
        _s3_body,
        grid=(N // BLK,),
        in_specs=[pl.BlockSpec((BLK, 1), lambda i: (i, 0)), blk, blk, blk],
        out_specs=[out, out, out],
        out_shape=[jax.ShapeDtypeStruct((MPAD, D), jnp.float32)] * 3,
    )(ids, ex, fs, fc)
    return tuple(r[:M] for r in res)


WPAD = MPAD // 128


def _p1_body(ids_ref, w_ref, cnt_ref, wm_ref):
    i = pl.program_id(0)

    @pl.when(i == 0)
    def _():
        cnt_ref[...] = jnp.zeros_like(cnt_ref)
        wm_ref[...] = jnp.full_like(wm_ref, -3.4e38)

    ids = ids_ref[...]
    wv = w_ref[...]
    w_lo = ids_ref[0, 0] // 128
    w_hi = ids_ref[BLK - 1, 0] // 128
    cols = lax.broadcasted_iota(jnp.int32, (1, 128), 1)

    def win(w, _):
        oh = ids == w * 128 + cols
        cnt_ref[pl.ds(w, 1), :] += jnp.sum(oh.astype(jnp.float32), axis=0,
                                           keepdims=True)
        vals = jnp.where(oh, wv, -3.4e38)
        wm_ref[pl.ds(w, 1), :] = jnp.maximum(
            wm_ref[pl.ds(w, 1), :], jnp.max(vals, axis=0, keepdims=True))
        return None

    lax.fori_loop(w_lo, w_hi + 1, win, None)


def _p1(ids, weight):
    out = pl.BlockSpec((WPAD, 128), lambda i: (0, 0))
    return pl.pallas_call(
        _p1_body,
        grid=(N // BLK,),
        in_specs=[pl.BlockSpec((BLK, 1), lambda i: (i, 0))] * 2,
        out_specs=[out, out],
        out_shape=[jax.ShapeDtypeStruct((WPAD, 128), jnp.float32)] * 2,
    )(ids, weight)


def _p2_body(ids_ref, w_ref, wm_ref, e_ref, den_ref):
    i = pl.program_id(0)

    @pl.when(i == 0)
    def _():
        den_ref[...] = jnp.zeros_like(den_ref)

    ids = ids_ref[...]
    wv = w_ref[...]
    w_lo = ids_ref[0, 0] // 128
    w_hi = ids_ref[BLK - 1, 0] // 128
    cols = lax.broadcasted_iota(jnp.int32, (1, 128), 1)

    def win1(w, g):
        oh = (ids == w * 128 + cols).astype(jnp.float32)
        return g + jnp.sum(oh * wm_ref[pl.ds(w, 1), :], axis=1, keepdims=True)

    gath = lax.fori_loop(w_lo, w_hi + 1, win1,
                         jnp.zeros((BLK, 1), jnp.float32))
    e = jnp.exp(wv - gath)
    e_ref[...] = e

    def win2(w, _):
        oh = (ids == w * 128 + cols).astype(jnp.float32)
        den_ref[pl.ds(w, 1), :] += jnp.sum(oh * e, axis=0, keepdims=True)
        return None

    lax.fori_loop(w_lo, w_hi + 1, win2, None)


def _p2(ids, weight, wm):
    return pl.pallas_call(
        _p2_body,
        grid=(N // BLK,),
        in_specs=[pl.BlockSpec((BLK, 1), lambda i: (i, 0))] * 2 +
                 [pl.BlockSpec((WPAD, 128), lambda i: (0, 0))],
        out_specs=[pl.BlockSpec((BLK, 1), lambda i: (i, 0)),
                   pl.BlockSpec((WPAD, 128), lambda i: (0, 0))],
        out_shape=[jax.ShapeDtypeStruct((N, 1), jnp.float32),
                   jax.ShapeDtypeStruct((WPAD, 128), jnp.float32)],
    )(ids, weight, wm)


def kernel(inputs, unq_inv, W_lin, b_lin, g_norm, be_norm, W_w1, b_w1, g_w, be_w, W_w2, b_w2, W_pm, b_pm, g_pm, be_pm, W_pw, b_pw, g_lk, be_lk):
    r2 = lambda v: v.reshape(1, -1)
    A, s1 = _k1(inputs, W_lin, r2(b_lin))
    x, B, C, s2, s3 = _k2(A, s1, r2(g_norm), r2(be_norm), W_w1, r2(b_w1), W_pm, r2(b_pm))
    weight = _k3a(B, s2, r2(g_w), r2(be_w), W_w2, b_w2.reshape(1, 1))

    ids2 = unq_inv.astype(jnp.int32).reshape(-1, 1)
    cntw, wmw = _p1(ids2, weight)
    e2, denw = _p2(ids2, weight, wmw)
    counts = cntw.reshape(-1)[:M]
    denom = denw.reshape(-1)[:M]
    inv_scale = jnp.where(counts > 0, 1.0 / (denom * jnp.clip(counts, 1.0, None)), 0.0)
    cntpos = (counts > 0).astype(jnp.float32)

    fs, fc, ex = _k3b(C, x, e2, s3, r2(g_pm), r2(be_pm), W_pw, b_pw.reshape(1, -1))

    sum_ex, add_s, add_c = _s3(ids2, ex, fs, fc)
    xmax = jax.ops.segment_max(x, unq_inv, num_segments=M)

    mean_feat, Dm, s4 = _k4(sum_ex, inv_scale[:, None], W_pm, r2(b_pm))
    final, s5 = _k5(Dm, s4, r2(g_pm), r2(be_pm), W_pw, r2(b_pw), add_s, add_c)
    return _k6(final, s5, r2(g_lk), r2(be_lk), mean_feat, xmax, cntpos[:, None])
